# baseline (speedup 1.0000x reference)
import functools

import jax
import jax.numpy as jnp
from jax import lax
from jax.experimental import pallas as pl
from jax.experimental.pallas import tpu as pltpu
from jax.experimental.pallas import tpu_sc as plsc

N = 10000
C = 128
P = 320000
E = 5000
TAU = 1.0

NC = 2
NS = 16
NW = NC * NS
K = 80
SUP = 8
SUPERS = 16
CHUNKS = SUP * SUPERS
PER_W = CHUNKS * K
PPAD = NW * PER_W - P
W = 2 * C
EP = 5120
ROWS_PER_TILE = EP // NS
OB = 32
L = 16
HB = EP // 128


def _table_body(s_ref, t_ref, o_ref):
    def softmax(x):
        m = jnp.max(x, axis=1, keepdims=True)
        ex = jnp.exp(x - m)
        return ex / jnp.sum(ex, axis=1, keepdims=True)

    o_ref[...] = jnp.concatenate([softmax(s_ref[...]), softmax(t_ref[...])],
                                 axis=1)


def _build_table(pred_s, pred_t):
    R = 400
    return pl.pallas_call(
        _table_body,
        grid=(N // R,),
        in_specs=[pl.BlockSpec((R, C), lambda i: (i, 0)),
                  pl.BlockSpec((R, C), lambda i: (i, 0))],
        out_specs=pl.BlockSpec((R, W), lambda i: (i, 0)),
        out_shape=jax.ShapeDtypeStruct((N, W), jnp.float32),
    )(pred_s, pred_t)


def _sc_body(table_hbm, vidx_hbm, eidx_hbm, out_hbm,
             iv0, iv1, ie0, ie1, rows0, rows1, acc_sh,
             sem_ix, sem_g0, sem_g1, sem_s0, sem_s1):
    cid = lax.axis_index("c")
    sid = lax.axis_index("s")
    wid = sid * NC + cid
    rows = (rows0, rows1)
    sem_g = (sem_g0, sem_g1)
    sem_s = (sem_s0, sem_s1)
    NT = SUPERS // 2

    il0 = pltpu.async_copy(vidx_hbm.at[wid, 0], iv0, sem_ix)
    il1 = pltpu.async_copy(eidx_hbm.at[wid, 0], ie0, sem_ix)

    zero = jnp.zeros((L,), jnp.float32)
    stage_v = rows0.at[pl.ds(0, OB)]

    def zstore(i, carry):
        r = i // (W // L)
        c = i % (W // L)
        rows0[r, pl.ds(c * L, L)] = zero
        return carry

    lax.fori_loop(0, OB * (W // L), zstore, 0)

    def zcopy(j, carry):
        r0 = sid * ROWS_PER_TILE + j * OB
        pltpu.sync_copy(stage_v, acc_sh.at[pl.ds(r0, OB)])
        return carry

    lax.fori_loop(0, ROWS_PER_TILE // OB, zcopy, 0)
    il0.wait()
    il1.wait()
    plsc.subcore_barrier()

    def gather(ivb, k, b):
        pltpu.async_copy(table_hbm.at[ivb.at[k]], rows[b], sem_g[b])

    def wait_g(b):
        pltpu.make_async_copy(table_hbm.at[iv0.at[0]], rows[b], sem_g[b]).wait()

    def scatter(ieb, k, b):
        pltpu.async_copy(rows[b], acc_sh.at[ieb.at[k]], sem_s[b], add=True)

    def wait_s(b):
        pltpu.make_async_copy(rows[b], acc_sh.at[ie0.at[0]], sem_s[b]).wait()

    gather(iv0, 0, 0)

    def super_pair(t, carry):
        for half in range(2):
            iv_cur, ie_cur = (iv0, ie0) if half == 0 else (iv1, ie1)
            iv_nxt, ie_nxt = (iv1, ie1) if half == 0 else (iv0, ie0)
            s = 2 * t + half
            if half == 0:
                pltpu.async_copy(vidx_hbm.at[wid, s + 1], iv_nxt, sem_ix)
                pltpu.async_copy(eidx_hbm.at[wid, s + 1], ie_nxt, sem_ix)
            else:
                @pl.when(t + 1 < NT)
                def _():
                    pltpu.async_copy(vidx_hbm.at[wid, s + 1], iv_nxt, sem_ix)
                    pltpu.async_copy(eidx_hbm.at[wid, s + 1], ie_nxt, sem_ix)
            for k in range(SUP):
                b = k % 2
                wait_g(b)
                if k < SUP - 1:
                    nb = (k + 1) % 2
                    if half == 0 and k == 0:
                        @pl.when(t > 0)
                        def _():
                            wait_s(nb)
                    else:
                        wait_s(nb)
                    gather(iv_cur, k + 1, nb)
                else:
                    def _boundary():
                        pltpu.make_async_copy(vidx_hbm.at[wid, 0], iv_nxt,
                                              sem_ix).wait()
                        pltpu.make_async_copy(eidx_hbm.at[wid, 0], ie_nxt,
                                              sem_ix).wait()
                        wait_s(0)
                        gather(iv_nxt, 0, 0)
                    if half == 0:
                        _boundary()
                    else:
                        pl.when(t + 1 < NT)(_boundary)
                scatter(ie_cur, k, b)
        return carry

    lax.fori_loop(0, NT, super_pair, 0)
    wait_s(0)
    wait_s(1)
    plsc.subcore_barrier()

    def ocopy(j, carry):
        r0 = sid * ROWS_PER_TILE + j * OB
        pltpu.sync_copy(acc_sh.at[pl.ds(r0, OB)], stage_v)
        pltpu.sync_copy(stage_v, out_hbm.at[cid, pl.ds(r0, OB)])
        return carry

    lax.fori_loop(0, ROWS_PER_TILE // OB, ocopy, 0)


def _sc_aggregate(table, v_idx, e_idx):
    mesh = plsc.VectorSubcoreMesh(core_axis_name="c", subcore_axis_name="s")
    k = functools.partial(
        pl.kernel,
        mesh=mesh,
        compiler_params=pltpu.CompilerParams(use_tc_tiling_on_sc=False),
        out_type=jax.ShapeDtypeStruct((NC, EP, W), jnp.float32),
        scratch_types=[
            pltpu.VMEM((SUP, K), jnp.int32),
            pltpu.VMEM((SUP, K), jnp.int32),
            pltpu.VMEM((SUP, K), jnp.int32),
            pltpu.VMEM((SUP, K), jnp.int32),
            pltpu.VMEM((K, W), jnp.float32),
            pltpu.VMEM((K, W), jnp.float32),
            pltpu.VMEM_SHARED((EP, W), jnp.float32),
            pltpu.SemaphoreType.DMA,
            pltpu.SemaphoreType.DMA,
            pltpu.SemaphoreType.DMA,
            pltpu.SemaphoreType.DMA,
            pltpu.SemaphoreType.DMA,
        ],
    )(_sc_body)
    return k(table,
             v_idx.reshape(NW, SUPERS, SUP, K),
             e_idx.reshape(NW, SUPERS, SUP, K))


PB = 12800
GC = P // PB


def _counts_body(e_ref, o_ref):
    i = pl.program_id(0)

    @pl.when(i == 0)
    def _():
        o_ref[...] = jnp.zeros((HB, C), jnp.float32)

    x = e_ref[0]
    hi = x // C
    lo = x - hi * C
    oh_hi = (jnp.broadcast_to(hi, (HB, PB))
             == lax.broadcasted_iota(jnp.int32, (HB, PB), 0)
             ).astype(jnp.float32)
    oh_lo = (jnp.broadcast_to(lo, (C, PB))
             == lax.broadcasted_iota(jnp.int32, (C, PB), 0)
             ).astype(jnp.float32)
    o_ref[...] += lax.dot_general(oh_hi, oh_lo, (((1,), (1,)), ((), ())),
                                  preferred_element_type=jnp.float32)


def _edge_counts(e_idx):
    e3 = e_idx.reshape(GC, 1, PB)
    return pl.pallas_call(
        _counts_body,
        grid=(GC,),
        in_specs=[pl.BlockSpec((1, 1, PB), lambda i: (i, 0, 0))],
        out_specs=pl.BlockSpec((HB, C), lambda i: (0, 0)),
        out_shape=jax.ShapeDtypeStruct((HB, C), jnp.float32),
    )(e3)


R3 = 200
G3 = E // R3


def _loss_body(parts_ref, cnt_ref, delta_ref, u_ref, o_ref, acc):
    i = pl.program_id(0)

    @pl.when(i == 0)
    def _():
        acc[0] = 0.0
        acc[1] = 0.0

    x = parts_ref[0] + parts_ref[1]
    cnt = jnp.maximum(cnt_ref[...], 1.0)
    mean_s = x[:, :C] / cnt
    mean_t = x[:, C:] / cnt
    log_inp = jnp.log(mean_s / TAU + 1e-09)
    tgt = mean_t / TAU
    per_edge = jnp.sum(tgt * (jnp.log(tgt) - log_inp), axis=1, keepdims=True)

    p = jnp.clip(delta_ref[...], 0.0, 1.0)
    maskf = (u_ref[...] < p).astype(jnp.float32)
    acc[0] += jnp.sum(maskf * per_edge)
    acc[1] += jnp.sum(maskf)

    @pl.when(i == G3 - 1)
    def _():
        n = acc[1]
        loss = acc[0] / jnp.maximum(n, 1.0)
        o_ref[...] = jnp.full((1, 1), jnp.where(n > 0.0, loss, 0.0),
                              dtype=jnp.float32)


def _finalize(parts, cnt_col, delta_col, u_col):
    out = pl.pallas_call(
        _loss_body,
        grid=(G3,),
        in_specs=[pl.BlockSpec((NC, R3, W), lambda i: (0, i, 0)),
                  pl.BlockSpec((R3, 1), lambda i: (i, 0)),
                  pl.BlockSpec((R3, 1), lambda i: (i, 0)),
                  pl.BlockSpec((R3, 1), lambda i: (i, 0))],
        out_specs=pl.BlockSpec((1, 1), lambda i: (0, 0)),
        out_shape=jax.ShapeDtypeStruct((1, 1), jnp.float32),
        scratch_shapes=[pltpu.SMEM((2,), jnp.float32)],
    )(parts, cnt_col, delta_col, u_col)
    return out[0, 0]


def kernel(pred_s, pred_t, delta_e_, v_idx, e_idx):
    table = _build_table(pred_s, pred_t)
    v_pad = jnp.concatenate([v_idx, jnp.zeros((PPAD,), jnp.int32)])
    e_pad = jnp.concatenate([e_idx, jnp.full((PPAD,), EP - 1, jnp.int32)])
    parts = _sc_aggregate(table, v_pad, e_pad)
    counts = _edge_counts(e_idx).reshape(EP, 1)[:E]
    u = jax.random.uniform(jax.random.key(42), (E,), jnp.float32)
    return _finalize(parts, counts, delta_e_[:, None], u[:, None])


# speedup vs baseline: 2.6569x; 2.6569x over previous
import functools

import jax
import jax.numpy as jnp
from jax import lax
from jax.experimental import pallas as pl
from jax.experimental.pallas import tpu as pltpu
from jax.experimental.pallas import tpu_sc as plsc

N = 10000
C = 128
P = 320000
E = 5000
TAU = 1.0

NC = 2
NS = 16
NW = NC * NS
K = 40
NB = 4
SUP = 32
SUPERS = 8
CHUNKS = SUP * SUPERS
PER_W = CHUNKS * K
PPAD = NW * PER_W - P
W = 2 * C
EP = 5120
ROWS_PER_TILE = EP // NS
OB = 32
L = 16
HB = EP // 128


def _table_body(s_ref, t_ref, o_ref):
    def softmax(x):
        m = jnp.max(x, axis=1, keepdims=True)
        ex = jnp.exp(x - m)
        return ex / jnp.sum(ex, axis=1, keepdims=True)

    o_ref[...] = jnp.concatenate([softmax(s_ref[...]), softmax(t_ref[...])],
                                 axis=1)


def _build_table(pred_s, pred_t):
    R = 400
    return pl.pallas_call(
        _table_body,
        grid=(N // R,),
        in_specs=[pl.BlockSpec((R, C), lambda i: (i, 0)),
                  pl.BlockSpec((R, C), lambda i: (i, 0))],
        out_specs=pl.BlockSpec((R, W), lambda i: (i, 0)),
        out_shape=jax.ShapeDtypeStruct((N, W), jnp.float32),
    )(pred_s, pred_t)


def _sc_body(table_hbm, vidx_hbm, eidx_hbm, out_hbm,
             iv0, iv1, ie0, ie1, rows0, rows1, rows2, rows3, acc_sh,
             sem_ix, sem_g0, sem_g1, sem_g2, sem_g3,
             sem_s0, sem_s1, sem_s2, sem_s3):
    cid = lax.axis_index("c")
    sid = lax.axis_index("s")
    wid = sid * NC + cid
    rows = (rows0, rows1, rows2, rows3)
    sem_g = (sem_g0, sem_g1, sem_g2, sem_g3)
    sem_s = (sem_s0, sem_s1, sem_s2, sem_s3)
    NT = SUPERS // 2

    il0 = pltpu.async_copy(vidx_hbm.at[wid, 0], iv0, sem_ix)
    il1 = pltpu.async_copy(eidx_hbm.at[wid, 0], ie0, sem_ix)

    zero = jnp.zeros((L,), jnp.float32)
    stage_v = rows0.at[pl.ds(0, OB)]

    def zstore(i, carry):
        r = i // (W // L)
        c = i % (W // L)
        rows0[r, pl.ds(c * L, L)] = zero
        return carry

    lax.fori_loop(0, OB * (W // L), zstore, 0)

    def zcopy(j, carry):
        r0 = sid * ROWS_PER_TILE + j * OB
        pltpu.sync_copy(stage_v, acc_sh.at[pl.ds(r0, OB)])
        return carry

    lax.fori_loop(0, ROWS_PER_TILE // OB, zcopy, 0)
    il0.wait()
    il1.wait()
    plsc.subcore_barrier()

    def gather(ivb, k, b):
        pltpu.async_copy(table_hbm.at[ivb.at[k]], rows[b], sem_g[b])

    def wait_g(b):
        pltpu.make_async_copy(table_hbm.at[iv0.at[0]], rows[b], sem_g[b]).wait()

    def scatter(ieb, k, b):
        pltpu.async_copy(rows[b], acc_sh.at[ieb.at[k]], sem_s[b], add=True)

    def wait_s(b):
        pltpu.make_async_copy(rows[b], acc_sh.at[ie0.at[0]], sem_s[b]).wait()

    for c in range(NB - 1):
        gather(iv0, c, c % NB)

    def super_pair(t, carry):
        for half in range(2):
            iv_cur, ie_cur = (iv0, ie0) if half == 0 else (iv1, ie1)
            iv_nxt, ie_nxt = (iv1, ie1) if half == 0 else (iv0, ie0)
            s = 2 * t + half
            if half == 0:
                pltpu.async_copy(vidx_hbm.at[wid, s + 1], iv_nxt, sem_ix)
                pltpu.async_copy(eidx_hbm.at[wid, s + 1], ie_nxt, sem_ix)
            else:
                @pl.when(t + 1 < NT)
                def _():
                    pltpu.async_copy(vidx_hbm.at[wid, s + 1], iv_nxt, sem_ix)
                    pltpu.async_copy(eidx_hbm.at[wid, s + 1], ie_nxt, sem_ix)
            for k in range(SUP):
                b = k % NB
                wait_g(b)
                scatter(ie_cur, k, b)
                b2 = (k + NB - 1) % NB

                def _wait_prev():
                    wait_s(b2)

                def _issue(ivb, kk):
                    gather(ivb, kk, b2)

                if k < SUP - (NB - 1):
                    if half == 0 and k == 0:
                        @pl.when(t > 0)
                        def _():
                            _wait_prev()
                    else:
                        _wait_prev()
                    _issue(iv_cur, k + NB - 1)
                else:
                    def _boundary():
                        if k == SUP - (NB - 1):
                            pltpu.make_async_copy(vidx_hbm.at[wid, 0], iv_nxt,
                                                  sem_ix).wait()
                            pltpu.make_async_copy(eidx_hbm.at[wid, 0], ie_nxt,
                                                  sem_ix).wait()
                        _wait_prev()
                        _issue(iv_nxt, k + NB - 1 - SUP)
                    if half == 0:
                        _boundary()
                    else:
                        pl.when(t + 1 < NT)(_boundary)
        return carry

    lax.fori_loop(0, NT, super_pair, 0)
    for b in range(NB):
        wait_s(b)
    plsc.subcore_barrier()

    def ocopy(j, carry):
        r0 = sid * ROWS_PER_TILE + j * OB
        pltpu.sync_copy(acc_sh.at[pl.ds(r0, OB)], stage_v)
        pltpu.sync_copy(stage_v, out_hbm.at[cid, pl.ds(r0, OB)])
        return carry

    lax.fori_loop(0, ROWS_PER_TILE // OB, ocopy, 0)


def _sc_aggregate(table, v_idx, e_idx):
    mesh = plsc.VectorSubcoreMesh(core_axis_name="c", subcore_axis_name="s")
    k = functools.partial(
        pl.kernel,
        mesh=mesh,
        compiler_params=pltpu.CompilerParams(use_tc_tiling_on_sc=False),
        out_type=jax.ShapeDtypeStruct((NC, EP, W), jnp.float32),
        scratch_types=(
            [pltpu.VMEM((SUP, K), jnp.int32)] * 4
            + [pltpu.VMEM((K, W), jnp.float32)] * NB
            + [pltpu.VMEM_SHARED((EP, W), jnp.float32)]
            + [pltpu.SemaphoreType.DMA] * (1 + 2 * NB)
        ),
    )(_sc_body)
    return k(table,
             v_idx.reshape(NW, SUPERS, SUP, K),
             e_idx.reshape(NW, SUPERS, SUP, K))


PB = 2560
GC = P // PB


def _counts_body(e_ref, o_ref):
    i = pl.program_id(0)

    @pl.when(i == 0)
    def _():
        o_ref[...] = jnp.zeros((HB, C), jnp.float32)

    x = e_ref[0]
    hi = x // C
    lo = x - hi * C
    oh_hi = (jnp.broadcast_to(hi, (HB, PB))
             == lax.broadcasted_iota(jnp.int32, (HB, PB), 0)
             ).astype(jnp.float32)
    oh_lo = (jnp.broadcast_to(lo, (C, PB))
             == lax.broadcasted_iota(jnp.int32, (C, PB), 0)
             ).astype(jnp.float32)
    o_ref[...] += lax.dot_general(oh_hi, oh_lo, (((1,), (1,)), ((), ())),
                                  preferred_element_type=jnp.float32)


def _edge_counts(e_idx):
    e3 = e_idx.reshape(GC, 1, PB)
    return pl.pallas_call(
        _counts_body,
        grid=(GC,),
        in_specs=[pl.BlockSpec((1, 1, PB), lambda i: (i, 0, 0))],
        out_specs=pl.BlockSpec((HB, C), lambda i: (0, 0)),
        out_shape=jax.ShapeDtypeStruct((HB, C), jnp.float32),
    )(e3)


R3 = 200
G3 = E // R3


def _loss_body(parts_ref, cnt_ref, delta_ref, u_ref, o_ref, acc):
    i = pl.program_id(0)

    @pl.when(i == 0)
    def _():
        acc[0] = 0.0
        acc[1] = 0.0

    x = parts_ref[0] + parts_ref[1]
    cnt = jnp.maximum(cnt_ref[...], 1.0)
    mean_s = x[:, :C] / cnt
    mean_t = x[:, C:] / cnt
    log_inp = jnp.log(mean_s / TAU + 1e-09)
    tgt = mean_t / TAU
    per_edge = jnp.sum(tgt * (jnp.log(tgt) - log_inp), axis=1, keepdims=True)

    p = jnp.clip(delta_ref[...], 0.0, 1.0)
    maskf = (u_ref[...] < p).astype(jnp.float32)
    acc[0] += jnp.sum(maskf * per_edge)
    acc[1] += jnp.sum(maskf)

    @pl.when(i == G3 - 1)
    def _():
        n = acc[1]
        loss = acc[0] / jnp.maximum(n, 1.0)
        o_ref[...] = jnp.full((1, 1), jnp.where(n > 0.0, loss, 0.0),
                              dtype=jnp.float32)


def _finalize(parts, cnt_col, delta_col, u_col):
    out = pl.pallas_call(
        _loss_body,
        grid=(G3,),
        in_specs=[pl.BlockSpec((NC, R3, W), lambda i: (0, i, 0)),
                  pl.BlockSpec((R3, 1), lambda i: (i, 0)),
                  pl.BlockSpec((R3, 1), lambda i: (i, 0)),
                  pl.BlockSpec((R3, 1), lambda i: (i, 0))],
        out_specs=pl.BlockSpec((1, 1), lambda i: (0, 0)),
        out_shape=jax.ShapeDtypeStruct((1, 1), jnp.float32),
        scratch_shapes=[pltpu.SMEM((2,), jnp.float32)],
    )(parts, cnt_col, delta_col, u_col)
    return out[0, 0]


def kernel(pred_s, pred_t, delta_e_, v_idx, e_idx):
    table = _build_table(pred_s, pred_t)
    v_junk = jnp.arange(PPAD, dtype=jnp.int32) % N
    v_pad = jnp.concatenate([v_idx, v_junk])
    e_junk = E + (jnp.arange(PPAD, dtype=jnp.int32) % (EP - E))
    e_pad = jnp.concatenate([e_idx, e_junk])
    parts = _sc_aggregate(table, v_pad, e_pad)
    counts = _edge_counts(e_idx).reshape(EP, 1)[:E]
    u = jax.random.uniform(jax.random.key(42), (E,), jnp.float32)
    return _finalize(parts, counts, delta_e_[:, None], u[:, None])
